# baseline (speedup 1.0000x reference)
import jax
import jax.numpy as jnp
from jax.experimental import pallas as pl
from jax.experimental.pallas import tpu as pltpu

VOCAB = 32000
ROW_BLK = 128


def _onehot_kernel(ids_ref, out_ref):
    next_id = jax.lax.rem(ids_ref[...] + 1, VOCAB)
    col = jax.lax.broadcasted_iota(jnp.int32, out_ref.shape, 1)
    out_ref[...] = jnp.where(col == next_id, 20.0, -20.0)


def kernel(input_ids):
    B, S = input_ids.shape
    rows = B * S
    ids = input_ids.reshape(rows, 1).astype(jnp.int32)
    out = pl.pallas_call(
        _onehot_kernel,
        grid=(rows // ROW_BLK,),
        in_specs=[pl.BlockSpec((ROW_BLK, 1), lambda i: (i, 0))],
        out_specs=pl.BlockSpec((ROW_BLK, VOCAB), lambda i: (i, 0)),
        out_shape=jax.ShapeDtypeStruct((rows, VOCAB), jnp.float32),
        compiler_params=pltpu.CompilerParams(
            dimension_semantics=("parallel",),
        ),
    )(ids)
    return out.reshape(B, S, VOCAB)


# speedup vs baseline: 1.0783x; 1.0783x over previous
import jax
import jax.numpy as jnp
from jax.experimental import pallas as pl
from jax.experimental.pallas import tpu as pltpu

VOCAB = 32000
ROW_BLK = 32


def _onehot_kernel(ids_ref, out_ref):
    next_id = jax.lax.rem(ids_ref[...] + 1, VOCAB)
    col = jax.lax.broadcasted_iota(jnp.int32, out_ref.shape, 1)
    out_ref[...] = jnp.where(col == next_id, 20.0, -20.0)


def kernel(input_ids):
    B, S = input_ids.shape
    rows = B * S
    ids = input_ids.reshape(rows, 1).astype(jnp.int32)
    out = pl.pallas_call(
        _onehot_kernel,
        grid=(rows // ROW_BLK,),
        in_specs=[pl.BlockSpec((ROW_BLK, 1), lambda i: (i, 0))],
        out_specs=pl.BlockSpec((ROW_BLK, VOCAB), lambda i: (i, 0)),
        out_shape=jax.ShapeDtypeStruct((rows, VOCAB), jnp.float32),
        compiler_params=pltpu.CompilerParams(
            dimension_semantics=("parallel",),
        ),
    )(ids)
    return out.reshape(B, S, VOCAB)
